# baseline (speedup 1.0000x reference)
import functools

import jax
import jax.numpy as jnp
from jax import lax
from jax.experimental import pallas as pl
from jax.experimental.pallas import tpu as pltpu
from jax.experimental.pallas import tpu_sc as plsc

N_NODES = 10000
D = 128
E = 320000

NC = 2
NS = 16
NW = NC * NS
L = 16

EPW = E // NW
C = 80
NCHUNK_W = EPW // C
G = C // L

_mesh = plsc.VectorSubcoreMesh(core_axis_name="c", subcore_axis_name="s")


@functools.partial(
    pl.kernel,
    mesh=_mesh,
    compiler_params=pltpu.CompilerParams(needs_layout_passes=False),
    out_type=jax.ShapeDtypeStruct((E,), jnp.float32),
    scratch_types=[
        pltpu.VMEM((EPW,), jnp.int32),
        pltpu.VMEM((EPW,), jnp.int32),
        pltpu.VMEM((2, C, D), jnp.float32),
        pltpu.VMEM((2, C, D), jnp.float32),
        pltpu.VMEM((2, C), jnp.float32),
        pltpu.VMEM((2, C), jnp.float32),
        pltpu.VMEM((C,), jnp.float32),
        pltpu.SemaphoreType.DMA,
        pltpu.SemaphoreType.DMA,
    ],
)
def _score_kernel(h_hbm, src_hbm, dst_hbm, bias_hbm, out_hbm,
                  sidx, didx, srows, drows, bsv, bdv, outv, sem0, sem1):
    sid = lax.axis_index("s")
    wid = sid * NC + lax.axis_index("c")
    base = wid * EPW
    pltpu.sync_copy(src_hbm.at[pl.ds(base, EPW)], sidx)
    pltpu.sync_copy(dst_hbm.at[pl.ds(base, EPW)], didx)
    sems = (sem0, sem1)

    def issue(t, b):
        off = pl.multiple_of(t * C, C)
        sem = sems[b]
        si = sidx.at[pl.ds(off, C)]
        di = didx.at[pl.ds(off, C)]
        pltpu.async_copy(h_hbm.at[si], srows.at[b], sem)
        pltpu.async_copy(h_hbm.at[di], drows.at[b], sem)
        pltpu.async_copy(bias_hbm.at[si], bsv.at[b], sem)
        pltpu.async_copy(bias_hbm.at[di], bdv.at[b], sem)

    def drain(b):
        sem = sems[b]
        pltpu.make_async_copy(h_hbm.at[pl.ds(0, C)], srows.at[b], sem).wait()
        pltpu.make_async_copy(h_hbm.at[pl.ds(0, C)], drows.at[b], sem).wait()
        pltpu.make_async_copy(bias_hbm.at[pl.ds(0, C)], bsv.at[b], sem).wait()
        pltpu.make_async_copy(bias_hbm.at[pl.ds(0, C)], bdv.at[b], sem).wait()

    lane = lax.iota(jnp.int32, L)
    onehot = [lane == e for e in range(L)]

    def compute(t, b):
        drain(b)

        def group_body(g, _):
            gb = pl.multiple_of(g * L, L)
            res = bsv[b, pl.ds(gb, L)] + bdv[b, pl.ds(gb, L)]
            for e in range(L):
                r = gb + e
                acc = srows[b, r, pl.ds(0, L)] * drows[b, r, pl.ds(0, L)]
                for k in range(1, D // L):
                    acc = acc + srows[b, r, pl.ds(k * L, L)] * drows[b, r, pl.ds(k * L, L)]
                res = jnp.where(onehot[e], res + jnp.sum(acc), res)
            outv[pl.ds(gb, L)] = res
            return 0

        lax.fori_loop(0, G, group_body, 0)
        pltpu.sync_copy(outv, out_hbm.at[pl.ds(base + t * C, C)])

    issue(0, 0)

    def pair_body(i, _):
        t = i * 2
        issue(t + 1, 1)
        compute(t, 0)
        issue(t + 2, 0)
        compute(t + 1, 1)
        return 0

    lax.fori_loop(0, (NCHUNK_W - 1) // 2, pair_body, 0)
    compute(NCHUNK_W - 1, 0)


def kernel(h, edge_index, bias):
    src = edge_index[0].astype(jnp.int32)
    dst = edge_index[1].astype(jnp.int32)
    return _score_kernel(h, src, dst, bias)


# speedup vs baseline: 1.2072x; 1.2072x over previous
import functools

import jax
import jax.numpy as jnp
from jax import lax
from jax.experimental import pallas as pl
from jax.experimental.pallas import tpu as pltpu
from jax.experimental.pallas import tpu_sc as plsc

N_NODES = 10000
D = 128
E = 320000

NC = 2
NS = 16
NW = NC * NS
L = 16

W = D // 2
EPW = E // NW
C = 80
NCHUNK_W = EPW // C
G = C // L

_mesh = plsc.VectorSubcoreMesh(core_axis_name="c", subcore_axis_name="s")


@functools.partial(
    pl.kernel,
    mesh=_mesh,
    compiler_params=pltpu.CompilerParams(
        needs_layout_passes=False, use_tc_tiling_on_sc=False),
    out_type=jax.ShapeDtypeStruct((E,), jnp.float32),
    scratch_types=[
        pltpu.VMEM((EPW,), jnp.int32),
        pltpu.VMEM((EPW,), jnp.int32),
        pltpu.VMEM((2, C, W), jnp.int32),
        pltpu.VMEM((2, C, W), jnp.int32),
        pltpu.VMEM((2, C), jnp.float32),
        pltpu.VMEM((2, C), jnp.float32),
        pltpu.VMEM((C,), jnp.float32),
        pltpu.SemaphoreType.DMA,
        pltpu.SemaphoreType.DMA,
    ],
)
def _score_kernel(hpk_hbm, src_hbm, dst_hbm, bias_hbm, out_hbm,
                  sidx, didx, srows, drows, bsv, bdv, outv, sem0, sem1):
    sid = lax.axis_index("s")
    wid = sid * NC + lax.axis_index("c")
    base = wid * EPW
    pltpu.sync_copy(src_hbm.at[pl.ds(base, EPW)], sidx)
    pltpu.sync_copy(dst_hbm.at[pl.ds(base, EPW)], didx)
    sems = (sem0, sem1)

    def issue(t, b):
        off = pl.multiple_of(t * C, C)
        sem = sems[b]
        si = sidx.at[pl.ds(off, C)]
        di = didx.at[pl.ds(off, C)]
        pltpu.async_copy(hpk_hbm.at[si], srows.at[b], sem)
        pltpu.async_copy(hpk_hbm.at[di], drows.at[b], sem)
        pltpu.async_copy(bias_hbm.at[si], bsv.at[b], sem)
        pltpu.async_copy(bias_hbm.at[di], bdv.at[b], sem)

    def drain(b):
        sem = sems[b]
        pltpu.make_async_copy(hpk_hbm.at[pl.ds(0, C)], srows.at[b], sem).wait()
        pltpu.make_async_copy(hpk_hbm.at[pl.ds(0, C)], drows.at[b], sem).wait()
        pltpu.make_async_copy(bias_hbm.at[pl.ds(0, C)], bsv.at[b], sem).wait()
        pltpu.make_async_copy(bias_hbm.at[pl.ds(0, C)], bdv.at[b], sem).wait()

    lane = lax.iota(jnp.int32, L)
    onehot = [lane == e for e in range(L)]

    def dot_step(b, r, k):
        vs = plsc.bitcast(srows[b, r, pl.ds(k * L, L)], jnp.bfloat16)
        vd = plsc.bitcast(drows[b, r, pl.ds(k * L, L)], jnp.bfloat16)
        s0, s1 = plsc.unpack(vs, format=plsc.PackFormat.INTERLEAVED)
        d0, d1 = plsc.unpack(vd, format=plsc.PackFormat.INTERLEAVED)
        return s0 * d0 + s1 * d1

    def compute(t, b):
        drain(b)

        def group_body(g, _):
            gb = pl.multiple_of(g * L, L)
            res = bsv[b, pl.ds(gb, L)] + bdv[b, pl.ds(gb, L)]
            for e in range(L):
                r = gb + e
                acc = dot_step(b, r, 0)
                for k in range(1, W // L):
                    acc = acc + dot_step(b, r, k)
                res = jnp.where(onehot[e], res + jnp.sum(acc), res)
            outv[pl.ds(gb, L)] = res
            return 0

        lax.fori_loop(0, G, group_body, 0)
        pltpu.sync_copy(outv, out_hbm.at[pl.ds(base + t * C, C)])

    issue(0, 0)

    def pair_body(i, _):
        t = i * 2
        issue(t + 1, 1)
        compute(t, 0)
        issue(t + 2, 0)
        compute(t + 1, 1)
        return 0

    lax.fori_loop(0, (NCHUNK_W - 1) // 2, pair_body, 0)
    compute(NCHUNK_W - 1, 0)


def kernel(h, edge_index, bias):
    src = edge_index[0].astype(jnp.int32)
    dst = edge_index[1].astype(jnp.int32)
    h16 = h.astype(jnp.bfloat16)
    hpk = lax.bitcast_convert_type(h16.reshape(N_NODES, W, 2), jnp.int32)
    return _score_kernel(hpk, src, dst, bias)


# speedup vs baseline: 1.4910x; 1.2351x over previous
import functools

import jax
import jax.numpy as jnp
from jax import lax
from jax.experimental import pallas as pl
from jax.experimental.pallas import tpu as pltpu
from jax.experimental.pallas import tpu_sc as plsc

N_NODES = 10000
D = 128
E = 320000

NC = 2
NS = 16
NW = NC * NS
L = 16

W = D // 2
EPW = E // NW
C = 80
NCHUNK_W = EPW // C
G = C // L

_mesh = plsc.VectorSubcoreMesh(core_axis_name="c", subcore_axis_name="s")


@functools.partial(
    pl.kernel,
    mesh=_mesh,
    compiler_params=pltpu.CompilerParams(
        needs_layout_passes=False, use_tc_tiling_on_sc=False),
    out_type=jax.ShapeDtypeStruct((E,), jnp.float32),
    scratch_types=[
        pltpu.VMEM((EPW,), jnp.int32),
        pltpu.VMEM((EPW,), jnp.int32),
        pltpu.VMEM((N_NODES,), jnp.float32),
        pltpu.VMEM((2, C, W), jnp.int32),
        pltpu.VMEM((2, C, W), jnp.int32),
        pltpu.VMEM((2, C), jnp.float32),
        pltpu.SemaphoreType.DMA,
        pltpu.SemaphoreType.DMA,
        pltpu.SemaphoreType.DMA,
        pltpu.SemaphoreType.DMA,
    ],
)
def _score_kernel(hpk_hbm, src_hbm, dst_hbm, bias_hbm, out_hbm,
                  sidx, didx, biasv, srows, drows, outv,
                  sem0, sem1, semo0, semo1):
    sid = lax.axis_index("s")
    wid = sid * NC + lax.axis_index("c")
    base = wid * EPW
    pltpu.sync_copy(src_hbm.at[pl.ds(base, EPW)], sidx)
    pltpu.sync_copy(dst_hbm.at[pl.ds(base, EPW)], didx)
    pltpu.sync_copy(bias_hbm, biasv)
    sems = (sem0, sem1)
    semos = (semo0, semo1)

    def issue(t, b):
        off = pl.multiple_of(t * C, C)
        sem = sems[b]
        pltpu.async_copy(hpk_hbm.at[sidx.at[pl.ds(off, C)]], srows.at[b], sem)
        pltpu.async_copy(hpk_hbm.at[didx.at[pl.ds(off, C)]], drows.at[b], sem)

    def drain(b):
        sem = sems[b]
        pltpu.make_async_copy(hpk_hbm.at[pl.ds(0, C)], srows.at[b], sem).wait()
        pltpu.make_async_copy(hpk_hbm.at[pl.ds(0, C)], drows.at[b], sem).wait()

    lane = lax.iota(jnp.int32, L)
    onehot = [lane == e for e in range(L)]

    def dot_step(b, r, k):
        vs = plsc.bitcast(srows[b, r, pl.ds(k * L, L)], jnp.bfloat16)
        vd = plsc.bitcast(drows[b, r, pl.ds(k * L, L)], jnp.bfloat16)
        p0, p1 = plsc.unpack(vs * vd, format=plsc.PackFormat.INTERLEAVED)
        return p0 + p1

    def compute(t, b):
        drain(b)
        @pl.when(t >= 2)
        def _drain_out():
            pltpu.make_async_copy(outv.at[b], out_hbm.at[pl.ds(0, C)],
                                  semos[b]).wait()

        def group_body(g, _):
            gb = pl.multiple_of(g * L, L)
            go = pl.multiple_of(t * C, L) + gb
            iv_s = sidx[pl.ds(go, L)]
            iv_d = didx[pl.ds(go, L)]
            res = plsc.load_gather(biasv, [iv_s]) + plsc.load_gather(biasv, [iv_d])
            for e in range(L):
                r = gb + e
                acc = dot_step(b, r, 0)
                for k in range(1, W // L):
                    acc = acc + dot_step(b, r, k)
                res = jnp.where(onehot[e], res + jnp.sum(acc), res)
            outv[b, pl.ds(gb, L)] = res
            return 0

        lax.fori_loop(0, G, group_body, 0)
        pltpu.async_copy(outv.at[b], out_hbm.at[pl.ds(base + t * C, C)], semos[b])

    issue(0, 0)

    def pair_body(i, _):
        t = i * 2
        issue(t + 1, 1)
        compute(t, 0)
        issue(t + 2, 0)
        compute(t + 1, 1)
        return 0

    lax.fori_loop(0, (NCHUNK_W - 1) // 2, pair_body, 0)
    compute(NCHUNK_W - 1, 0)
    pltpu.make_async_copy(outv.at[1], out_hbm.at[pl.ds(0, C)], semos[1]).wait()
    pltpu.make_async_copy(outv.at[0], out_hbm.at[pl.ds(0, C)], semos[0]).wait()


def kernel(h, edge_index, bias):
    src = edge_index[0].astype(jnp.int32)
    dst = edge_index[1].astype(jnp.int32)
    h16 = h.astype(jnp.bfloat16)
    hpk = lax.bitcast_convert_type(h16.reshape(N_NODES, W, 2), jnp.int32)
    return _score_kernel(hpk, src, dst, bias)


# speedup vs baseline: 1.8682x; 1.2530x over previous
import functools

import jax
import jax.numpy as jnp
from jax import lax
from jax.experimental import pallas as pl
from jax.experimental.pallas import tpu as pltpu
from jax.experimental.pallas import tpu_sc as plsc

N_NODES = 10000
D = 128
E = 320000

NC = 2
NS = 16
NW = NC * NS
L = 16
L2 = 32

EPW = E // NW
C = 80
NCHUNK_W = EPW // C
G = C // L

_mesh = plsc.VectorSubcoreMesh(core_axis_name="c", subcore_axis_name="s")


@functools.partial(
    pl.kernel,
    mesh=_mesh,
    compiler_params=pltpu.CompilerParams(
        needs_layout_passes=False, use_tc_tiling_on_sc=False),
    out_type=jax.ShapeDtypeStruct((E,), jnp.float32),
    scratch_types=[
        pltpu.VMEM((EPW,), jnp.int32),
        pltpu.VMEM((EPW,), jnp.int32),
        pltpu.VMEM((N_NODES,), jnp.float32),
        pltpu.VMEM((2, C, D), jnp.bfloat16),
        pltpu.VMEM((2, C, D), jnp.bfloat16),
        pltpu.VMEM((2, C), jnp.float32),
        pltpu.SemaphoreType.DMA,
        pltpu.SemaphoreType.DMA,
        pltpu.SemaphoreType.DMA,
        pltpu.SemaphoreType.DMA,
    ],
)
def _score_kernel(h16_hbm, ei_hbm, bias_hbm, out_hbm,
                  sidx, didx, biasv, srows, drows, outv,
                  sem0, sem1, semo0, semo1):
    sid = lax.axis_index("s")
    wid = sid * NC + lax.axis_index("c")
    base = wid * EPW
    pltpu.sync_copy(ei_hbm.at[0, pl.ds(base, EPW)], sidx)
    pltpu.sync_copy(ei_hbm.at[1, pl.ds(base, EPW)], didx)
    pltpu.sync_copy(bias_hbm, biasv)
    sems = (sem0, sem1)
    semos = (semo0, semo1)

    def issue(t, b):
        off = pl.multiple_of(t * C, C)
        sem = sems[b]
        pltpu.async_copy(h16_hbm.at[sidx.at[pl.ds(off, C)]], srows.at[b], sem)
        pltpu.async_copy(h16_hbm.at[didx.at[pl.ds(off, C)]], drows.at[b], sem)

    def drain(b):
        sem = sems[b]
        pltpu.make_async_copy(h16_hbm.at[pl.ds(0, C)], srows.at[b], sem).wait()
        pltpu.make_async_copy(h16_hbm.at[pl.ds(0, C)], drows.at[b], sem).wait()

    lane = lax.iota(jnp.int32, L)
    onehot = [lane == e for e in range(L)]

    def dot_step(b, r, k):
        vs = srows[b, r, pl.ds(k * L2, L2)]
        vd = drows[b, r, pl.ds(k * L2, L2)]
        p0, p1 = plsc.unpack(vs * vd, format=plsc.PackFormat.INTERLEAVED)
        return p0 + p1

    def compute(t, b):
        drain(b)
        @pl.when(t >= 2)
        def _drain_out():
            pltpu.make_async_copy(outv.at[b], out_hbm.at[pl.ds(0, C)],
                                  semos[b]).wait()

        def group_body(g, _):
            gb = pl.multiple_of(g * L, L)
            go = pl.multiple_of(t * C, L) + gb
            iv_s = sidx[pl.ds(go, L)]
            iv_d = didx[pl.ds(go, L)]
            res = plsc.load_gather(biasv, [iv_s]) + plsc.load_gather(biasv, [iv_d])
            for e in range(L):
                r = gb + e
                acc = dot_step(b, r, 0)
                for k in range(1, D // L2):
                    acc = acc + dot_step(b, r, k)
                res = jnp.where(onehot[e], res + jnp.sum(acc), res)
            outv[b, pl.ds(gb, L)] = res
            return 0

        lax.fori_loop(0, G, group_body, 0)
        pltpu.async_copy(outv.at[b], out_hbm.at[pl.ds(base + t * C, C)], semos[b])

    issue(0, 0)

    def pair_body(i, _):
        t = i * 2
        issue(t + 1, 1)
        compute(t, 0)
        issue(t + 2, 0)
        compute(t + 1, 1)
        return 0

    lax.fori_loop(0, (NCHUNK_W - 1) // 2, pair_body, 0)
    compute(NCHUNK_W - 1, 0)
    pltpu.make_async_copy(outv.at[1], out_hbm.at[pl.ds(0, C)], semos[1]).wait()
    pltpu.make_async_copy(outv.at[0], out_hbm.at[pl.ds(0, C)], semos[0]).wait()


def kernel(h, edge_index, bias):
    return _score_kernel(h.astype(jnp.bfloat16),
                         edge_index.astype(jnp.int32), bias)


# speedup vs baseline: 2.2669x; 1.2134x over previous
import functools

import jax
import jax.numpy as jnp
from jax import lax
from jax.experimental import pallas as pl
from jax.experimental.pallas import tpu as pltpu
from jax.experimental.pallas import tpu_sc as plsc

N_NODES = 10000
D = 128
E = 320000

NC = 2
NS = 16
NW = NC * NS
L = 16
L2 = 32

EPW = E // NW
C = 80
NCHUNK_W = EPW // C
G = C // L

_mesh = plsc.VectorSubcoreMesh(core_axis_name="c", subcore_axis_name="s")


@functools.partial(
    pl.kernel,
    mesh=_mesh,
    compiler_params=pltpu.CompilerParams(
        needs_layout_passes=False, use_tc_tiling_on_sc=False),
    out_type=jax.ShapeDtypeStruct((E,), jnp.float32),
    scratch_types=[
        pltpu.VMEM((EPW,), jnp.int32),
        pltpu.VMEM((EPW,), jnp.int32),
        pltpu.VMEM((N_NODES,), jnp.float32),
        pltpu.VMEM((3, C, D), jnp.bfloat16),
        pltpu.VMEM((3, C, D), jnp.bfloat16),
        pltpu.VMEM((3, C), jnp.float32),
        pltpu.SemaphoreType.DMA,
        pltpu.SemaphoreType.DMA,
        pltpu.SemaphoreType.DMA,
        pltpu.SemaphoreType.DMA,
        pltpu.SemaphoreType.DMA,
        pltpu.SemaphoreType.DMA,
    ],
)
def _score_kernel(h16_hbm, ei_hbm, bias_hbm, out_hbm,
                  sidx, didx, biasv, srows, drows, outv,
                  sem0, sem1, sem2, semo0, semo1, semo2):
    sid = lax.axis_index("s")
    wid = sid * NC + lax.axis_index("c")
    base = wid * EPW
    pltpu.sync_copy(ei_hbm.at[0, pl.ds(base, EPW)], sidx)
    pltpu.sync_copy(ei_hbm.at[1, pl.ds(base, EPW)], didx)
    pltpu.sync_copy(bias_hbm, biasv)
    sems = (sem0, sem1, sem2)
    semos = (semo0, semo1, semo2)

    def issue(t, b):
        off = pl.multiple_of(t * C, C)
        sem = sems[b]
        pltpu.async_copy(h16_hbm.at[sidx.at[pl.ds(off, C)]], srows.at[b], sem)
        pltpu.async_copy(h16_hbm.at[didx.at[pl.ds(off, C)]], drows.at[b], sem)

    def drain(b):
        sem = sems[b]
        pltpu.make_async_copy(h16_hbm.at[pl.ds(0, C)], srows.at[b], sem).wait()
        pltpu.make_async_copy(h16_hbm.at[pl.ds(0, C)], drows.at[b], sem).wait()

    lane = lax.iota(jnp.int32, L)
    onehot = [lane == e for e in range(L)]

    def dot_step(b, r, k):
        vs = srows[b, r, pl.ds(k * L2, L2)]
        vd = drows[b, r, pl.ds(k * L2, L2)]
        p0, p1 = plsc.unpack(vs * vd, format=plsc.PackFormat.INTERLEAVED)
        return p0 + p1

    def compute(t, b):
        drain(b)
        @pl.when(t >= 3)
        def _drain_out():
            pltpu.make_async_copy(outv.at[b], out_hbm.at[pl.ds(0, C)],
                                  semos[b]).wait()

        def group_body(g, _):
            gb = pl.multiple_of(g * L, L)
            go = pl.multiple_of(t * C, L) + gb
            iv_s = sidx[pl.ds(go, L)]
            iv_d = didx[pl.ds(go, L)]
            res = plsc.load_gather(biasv, [iv_s]) + plsc.load_gather(biasv, [iv_d])
            for e in range(L):
                r = gb + e
                acc = dot_step(b, r, 0)
                for k in range(1, D // L2):
                    acc = acc + dot_step(b, r, k)
                res = jnp.where(onehot[e], res + jnp.sum(acc), res)
            outv[b, pl.ds(gb, L)] = res
            return 0

        lax.fori_loop(0, G, group_body, 0)
        pltpu.async_copy(outv.at[b], out_hbm.at[pl.ds(base + t * C, C)], semos[b])

    issue(0, 0)
    issue(1, 1)

    def triple_body(i, _):
        t = i * 3
        issue(t + 2, 2)
        compute(t, 0)
        issue(t + 3, 0)
        compute(t + 1, 1)
        issue(t + 4, 1)
        compute(t + 2, 2)
        return 0

    lax.fori_loop(0, (NCHUNK_W - 2) // 3, triple_body, 0)
    compute(NCHUNK_W - 2, 0)
    compute(NCHUNK_W - 1, 1)
    pltpu.make_async_copy(outv.at[2], out_hbm.at[pl.ds(0, C)], semos[2]).wait()
    pltpu.make_async_copy(outv.at[1], out_hbm.at[pl.ds(0, C)], semos[1]).wait()
    pltpu.make_async_copy(outv.at[0], out_hbm.at[pl.ds(0, C)], semos[0]).wait()


def kernel(h, edge_index, bias):
    return _score_kernel(h.astype(jnp.bfloat16), edge_index, bias)


# speedup vs baseline: 2.3241x; 1.0252x over previous
import functools

import jax
import jax.numpy as jnp
from jax import lax
from jax.experimental import pallas as pl
from jax.experimental.pallas import tpu as pltpu
from jax.experimental.pallas import tpu_sc as plsc

N_NODES = 10000
D = 128
E = 320000

NC = 2
NS = 16
NW = NC * NS
L = 16
L2 = 32

EPW = E // NW
C = 112
NCHUNK_W = EPW // C
CT = EPW - NCHUNK_W * C
G = C // L
GT = CT // L

_mesh = plsc.VectorSubcoreMesh(core_axis_name="c", subcore_axis_name="s")


@functools.partial(
    pl.kernel,
    mesh=_mesh,
    compiler_params=pltpu.CompilerParams(
        needs_layout_passes=False, use_tc_tiling_on_sc=False),
    out_type=jax.ShapeDtypeStruct((E,), jnp.float32),
    scratch_types=[
        pltpu.VMEM((EPW,), jnp.int32),
        pltpu.VMEM((EPW,), jnp.int32),
        pltpu.VMEM((N_NODES,), jnp.float32),
        pltpu.VMEM((3, C, D), jnp.bfloat16),
        pltpu.VMEM((3, C, D), jnp.bfloat16),
        pltpu.VMEM((3, C), jnp.float32),
        pltpu.SemaphoreType.DMA,
        pltpu.SemaphoreType.DMA,
        pltpu.SemaphoreType.DMA,
        pltpu.SemaphoreType.DMA,
        pltpu.SemaphoreType.DMA,
        pltpu.SemaphoreType.DMA,
    ],
)
def _score_kernel(h16_hbm, ei_hbm, bias_hbm, out_hbm,
                  sidx, didx, biasv, srows, drows, outv,
                  sem0, sem1, sem2, semo0, semo1, semo2):
    sid = lax.axis_index("s")
    wid = sid * NC + lax.axis_index("c")
    base = wid * EPW
    pltpu.sync_copy(ei_hbm.at[0, pl.ds(base, EPW)], sidx)
    pltpu.sync_copy(ei_hbm.at[1, pl.ds(base, EPW)], didx)
    pltpu.sync_copy(bias_hbm, biasv)
    sems = (sem0, sem1, sem2)
    semos = (semo0, semo1, semo2)

    def issue(t, b):
        off = pl.multiple_of(t * C, C)
        sem = sems[b]
        pltpu.async_copy(h16_hbm.at[sidx.at[pl.ds(off, C)]], srows.at[b], sem)
        pltpu.async_copy(h16_hbm.at[didx.at[pl.ds(off, C)]], drows.at[b], sem)

    def drain(b):
        sem = sems[b]
        pltpu.make_async_copy(h16_hbm.at[pl.ds(0, C)], srows.at[b], sem).wait()
        pltpu.make_async_copy(h16_hbm.at[pl.ds(0, C)], drows.at[b], sem).wait()

    lane = lax.iota(jnp.int32, L)
    onehot = [lane == e for e in range(L)]

    def dot_step(b, r, k):
        vs = srows[b, r, pl.ds(k * L2, L2)]
        vd = drows[b, r, pl.ds(k * L2, L2)]
        p0, p1 = plsc.unpack(vs * vd, format=plsc.PackFormat.INTERLEAVED)
        return p0 + p1

    def compute(t, b):
        drain(b)
        @pl.when(t >= 3)
        def _drain_out():
            pltpu.make_async_copy(outv.at[b], out_hbm.at[pl.ds(0, C)],
                                  semos[b]).wait()

        def group_body(g, _):
            gb = pl.multiple_of(g * L, L)
            go = pl.multiple_of(t * C, L) + gb
            iv_s = sidx[pl.ds(go, L)]
            iv_d = didx[pl.ds(go, L)]
            res = plsc.load_gather(biasv, [iv_s]) + plsc.load_gather(biasv, [iv_d])
            for e in range(L):
                r = gb + e
                acc = dot_step(b, r, 0)
                for k in range(1, D // L2):
                    acc = acc + dot_step(b, r, k)
                res = jnp.where(onehot[e], res + jnp.sum(acc), res)
            outv[b, pl.ds(gb, L)] = res
            return 0

        lax.fori_loop(0, G, group_body, 0)
        pltpu.async_copy(outv.at[b], out_hbm.at[pl.ds(base + t * C, C)], semos[b])

    issue(0, 0)
    issue(1, 1)

    def triple_body(i, _):
        t = i * 3
        issue(t + 2, 2)
        compute(t, 0)
        issue(t + 3, 0)
        compute(t + 1, 1)
        issue(t + 4, 1)
        compute(t + 2, 2)
        return 0

    lax.fori_loop(0, (NCHUNK_W - 2) // 3, triple_body, 0)
    compute(NCHUNK_W - 2, 0)
    compute(NCHUNK_W - 1, 1)

    toff = NCHUNK_W * C
    pltpu.make_async_copy(outv.at[2], out_hbm.at[pl.ds(0, C)], semos[2]).wait()
    srt = srows.at[2].at[pl.ds(0, CT)]
    drt = drows.at[2].at[pl.ds(0, CT)]
    pltpu.async_copy(h16_hbm.at[sidx.at[pl.ds(toff, CT)]], srt, sem2)
    pltpu.async_copy(h16_hbm.at[didx.at[pl.ds(toff, CT)]], drt, sem2)
    pltpu.make_async_copy(h16_hbm.at[pl.ds(0, CT)], srt, sem2).wait()
    pltpu.make_async_copy(h16_hbm.at[pl.ds(0, CT)], drt, sem2).wait()
    for g in range(GT):
        gb = g * L
        go = toff + gb
        iv_s = sidx[pl.ds(go, L)]
        iv_d = didx[pl.ds(go, L)]
        res = plsc.load_gather(biasv, [iv_s]) + plsc.load_gather(biasv, [iv_d])
        for e in range(L):
            r = gb + e
            acc = dot_step(2, r, 0)
            for k in range(1, D // L2):
                acc = acc + dot_step(2, r, k)
            res = jnp.where(onehot[e], res + jnp.sum(acc), res)
        outv[2, pl.ds(gb, L)] = res
    pltpu.sync_copy(outv.at[2].at[pl.ds(0, CT)],
                    out_hbm.at[pl.ds(base + toff, CT)])

    pltpu.make_async_copy(outv.at[1], out_hbm.at[pl.ds(0, C)], semos[1]).wait()
    pltpu.make_async_copy(outv.at[0], out_hbm.at[pl.ds(0, C)], semos[0]).wait()


def kernel(h, edge_index, bias):
    return _score_kernel(h.astype(jnp.bfloat16), edge_index, bias)


# speedup vs baseline: 2.3434x; 1.0083x over previous
import functools

import jax
import jax.numpy as jnp
from jax import lax
from jax.experimental import pallas as pl
from jax.experimental.pallas import tpu as pltpu
from jax.experimental.pallas import tpu_sc as plsc

N_NODES = 10000
D = 128
E = 320000

NC = 2
NS = 16
NW = NC * NS
L = 16
L2 = 32

EPW = E // NW
C = 128
NCHUNK_W = EPW // C
CT = EPW - NCHUNK_W * C
G = C // L
GT = CT // L

_mesh = plsc.VectorSubcoreMesh(core_axis_name="c", subcore_axis_name="s")


@functools.partial(
    pl.kernel,
    mesh=_mesh,
    compiler_params=pltpu.CompilerParams(
        needs_layout_passes=False, use_tc_tiling_on_sc=False),
    out_type=jax.ShapeDtypeStruct((E,), jnp.float32),
    scratch_types=[
        pltpu.VMEM((EPW,), jnp.int32),
        pltpu.VMEM((EPW,), jnp.int32),
        pltpu.VMEM((N_NODES,), jnp.float32),
        pltpu.VMEM((3, C, D), jnp.bfloat16),
        pltpu.VMEM((3, C, D), jnp.bfloat16),
        pltpu.VMEM((3, C), jnp.float32),
        pltpu.SemaphoreType.DMA,
        pltpu.SemaphoreType.DMA,
        pltpu.SemaphoreType.DMA,
        pltpu.SemaphoreType.DMA,
        pltpu.SemaphoreType.DMA,
        pltpu.SemaphoreType.DMA,
    ],
)
def _score_kernel(h16_hbm, ei_hbm, bias_hbm, out_hbm,
                  sidx, didx, biasv, srows, drows, outv,
                  sem0, sem1, sem2, semo0, semo1, semo2):
    sid = lax.axis_index("s")
    wid = sid * NC + lax.axis_index("c")
    base = wid * EPW
    pltpu.sync_copy(ei_hbm.at[0, pl.ds(base, EPW)], sidx)
    pltpu.sync_copy(ei_hbm.at[1, pl.ds(base, EPW)], didx)
    pltpu.sync_copy(bias_hbm, biasv)
    sems = (sem0, sem1, sem2)
    semos = (semo0, semo1, semo2)

    def issue(t, b):
        off = pl.multiple_of(t * C, C)
        sem = sems[b]
        pltpu.async_copy(h16_hbm.at[sidx.at[pl.ds(off, C)]], srows.at[b], sem)
        pltpu.async_copy(h16_hbm.at[didx.at[pl.ds(off, C)]], drows.at[b], sem)

    def drain(b):
        sem = sems[b]
        pltpu.make_async_copy(h16_hbm.at[pl.ds(0, C)], srows.at[b], sem).wait()
        pltpu.make_async_copy(h16_hbm.at[pl.ds(0, C)], drows.at[b], sem).wait()

    lane = lax.iota(jnp.int32, L)
    onehot = [lane == e for e in range(L)]

    def dot_step(b, r, k):
        vs = srows[b, r, pl.ds(k * L2, L2)]
        vd = drows[b, r, pl.ds(k * L2, L2)]
        p0, p1 = plsc.unpack(vs * vd, format=plsc.PackFormat.INTERLEAVED)
        return p0 + p1

    def compute(t, b):
        drain(b)
        @pl.when(t >= 3)
        def _drain_out():
            pltpu.make_async_copy(outv.at[b], out_hbm.at[pl.ds(0, C)],
                                  semos[b]).wait()

        def group_body(g, _):
            gb = pl.multiple_of(g * L, L)
            go = pl.multiple_of(t * C, L) + gb
            iv_s = sidx[pl.ds(go, L)]
            iv_d = didx[pl.ds(go, L)]
            res = plsc.load_gather(biasv, [iv_s]) + plsc.load_gather(biasv, [iv_d])
            for e in range(L):
                r = gb + e
                acc = dot_step(b, r, 0)
                for k in range(1, D // L2):
                    acc = acc + dot_step(b, r, k)
                res = jnp.where(onehot[e], res + jnp.sum(acc), res)
            outv[b, pl.ds(gb, L)] = res
            return 0

        lax.fori_loop(0, G, group_body, 0)
        pltpu.async_copy(outv.at[b], out_hbm.at[pl.ds(base + t * C, C)], semos[b])

    issue(0, 0)
    issue(1, 1)

    def triple_body(i, _):
        t = i * 3
        issue(t + 2, 2)
        compute(t, 0)
        issue(t + 3, 0)
        compute(t + 1, 1)
        issue(t + 4, 1)
        compute(t + 2, 2)
        return 0

    lax.fori_loop(0, (NCHUNK_W - 3) // 3, triple_body, 0)
    issue(NCHUNK_W - 1, 2)
    compute(NCHUNK_W - 3, 0)
    compute(NCHUNK_W - 2, 1)
    compute(NCHUNK_W - 1, 2)

    toff = NCHUNK_W * C
    pltpu.make_async_copy(outv.at[2], out_hbm.at[pl.ds(0, C)], semos[2]).wait()
    srt = srows.at[2].at[pl.ds(0, CT)]
    drt = drows.at[2].at[pl.ds(0, CT)]
    pltpu.async_copy(h16_hbm.at[sidx.at[pl.ds(toff, CT)]], srt, sem2)
    pltpu.async_copy(h16_hbm.at[didx.at[pl.ds(toff, CT)]], drt, sem2)
    pltpu.make_async_copy(h16_hbm.at[pl.ds(0, CT)], srt, sem2).wait()
    pltpu.make_async_copy(h16_hbm.at[pl.ds(0, CT)], drt, sem2).wait()
    for g in range(GT):
        gb = g * L
        go = toff + gb
        iv_s = sidx[pl.ds(go, L)]
        iv_d = didx[pl.ds(go, L)]
        res = plsc.load_gather(biasv, [iv_s]) + plsc.load_gather(biasv, [iv_d])
        for e in range(L):
            r = gb + e
            acc = dot_step(2, r, 0)
            for k in range(1, D // L2):
                acc = acc + dot_step(2, r, k)
            res = jnp.where(onehot[e], res + jnp.sum(acc), res)
        outv[2, pl.ds(gb, L)] = res
    pltpu.sync_copy(outv.at[2].at[pl.ds(0, CT)],
                    out_hbm.at[pl.ds(base + toff, CT)])

    pltpu.make_async_copy(outv.at[1], out_hbm.at[pl.ds(0, C)], semos[1]).wait()
    pltpu.make_async_copy(outv.at[0], out_hbm.at[pl.ds(0, C)], semos[0]).wait()


def kernel(h, edge_index, bias):
    return _score_kernel(h.astype(jnp.bfloat16), edge_index, bias)
